# baseline (speedup 1.0000x reference)
import functools

import jax
import jax.numpy as jnp
from jax import lax
from jax.experimental import pallas as pl
from jax.experimental.pallas import tpu as pltpu

N = 10000
H = 128
O = 256
R = 16
E = 320000

BN = 400


def _dense_body(agg_ref, h_ref, w2_ref, wself_ref, bias_ref, out_ref):
    acc = jnp.dot(agg_ref[...], w2_ref[...], preferred_element_type=jnp.float32)
    acc += jnp.dot(h_ref[...], wself_ref[...], preferred_element_type=jnp.float32)
    out_ref[...] = jnp.maximum(acc + bias_ref[...], 0.0)


def _dense_stage(aggf, h, w2, w_self, bias2d):
    return pl.pallas_call(
        _dense_body,
        grid=(N // BN,),
        in_specs=[
            pl.BlockSpec((BN, R * H), lambda i: (i, 0)),
            pl.BlockSpec((BN, H), lambda i: (i, 0)),
            pl.BlockSpec((R * H, O), lambda i: (0, 0)),
            pl.BlockSpec((H, O), lambda i: (0, 0)),
            pl.BlockSpec((1, O), lambda i: (0, 0)),
        ],
        out_specs=pl.BlockSpec((BN, O), lambda i: (i, 0)),
        out_shape=jax.ShapeDtypeStruct((N, O), jnp.float32),
    )(aggf, h, w2, w_self, bias2d)


def kernel(h, edge_index, rel_type, norm, W, W_self, bias):
    src = edge_index[0].astype(jnp.int32)
    dst = edge_index[1].astype(jnp.int32)
    key = dst * R + rel_type.astype(jnp.int32)
    normf = norm[:, 0]

    agg = jnp.zeros((N * R, H), jnp.float32).at[key].add(h[src] * normf[:, None])

    aggf = agg.reshape(N, R * H)
    w2 = W.reshape(R * H, O)
    return _dense_stage(aggf, h, w2, W_self, bias.reshape(1, O))


# speedup vs baseline: 1.6812x; 1.6812x over previous
import functools

import jax
import jax.numpy as jnp
from jax import lax
from jax.experimental import pallas as pl
from jax.experimental.pallas import tpu as pltpu
from jax.experimental.pallas import tpu_sc as plsc

N = 10000
H = 128
O = 256
R = 16
E = 320000

NC = 2
NS = 16
L = 16

E_PAD = 327680
EPS = E_PAD // NS
SW = 2048
NSTRIPS = EPS // SW
SVREGS = SW // L

KP = 13440
PASSES = 6
ACC_ROWS = KP + 128
OUT_ROWS = PASSES * NC * KP
IDS_CAP = 2304
IDS_DUMP = 2176
BIGKEY = 10 ** 8

BN = 400


def _sc_body(h_hbm, keys_hbm, src_hbm, norm_hbm, agg_hbm,
             acc, keys_strip, ids_v, idxb, srcb, keyb, normb, lidxb, rows):
    c = lax.axis_index("c")
    s = lax.axis_index("s")
    ebase = s * EPS
    lane = lax.broadcasted_iota(jnp.int32, (L,), 0)
    zero16 = jnp.zeros((L,), jnp.float32)

    def pass_body(p, carry):
        kbase = p * (NC * KP) + c * KP

        def zloop(i, cz):
            r = i // 8
            jcol = (i % 8) * L
            rows[r, pl.ds(jcol, L)] = zero16
            return cz
        lax.fori_loop(0, 128 * 8, zloop, 0)
        sbase = s * (ACC_ROWS // NS)
        for t in range(6):
            pltpu.sync_copy(rows, acc.at[pl.ds(sbase + t * 128, 128)])
        pltpu.sync_copy(rows.at[pl.ds(0, 80)], acc.at[pl.ds(sbase + 768, 80)])
        plsc.subcore_barrier()

        def chunk(ch, cc):
            for j in range(8):
                idxb[pl.ds(j * L, L)] = ids_v[pl.ds(ch * 128 + j * L, L)]
            pltpu.sync_copy(src_hbm.at[idxb], srcb)
            pltpu.sync_copy(keys_hbm.at[idxb], keyb)
            pltpu.sync_copy(norm_hbm.at[idxb], normb)
            pltpu.sync_copy(h_hbm.at[srcb], rows)
            for j in range(8):
                kv = keyb[pl.ds(j * L, L)]
                rk = kv - kbase
                m = (rk >= 0) & (rk < KP)
                lidxb[pl.ds(j * L, L)] = jnp.where(m, rk, KP + lane + j * L % 128)
            def sloop(e, c2):
                nv = plsc.load_gather(normb, [jnp.broadcast_to(e, (L,))])
                for j in range(8):
                    rows[e, pl.ds(j * L, L)] = rows[e, pl.ds(j * L, L)] * nv
                return c2
            lax.fori_loop(0, 128, sloop, 0)
            pltpu.sync_copy(rows, acc.at[lidxb], add=True)
            return cc

        def strip_body(t, cnt):
            pltpu.sync_copy(keys_hbm.at[pl.ds(ebase + t * SW, SW)], keys_strip)
            def fv(v, cn):
                kv = keys_strip[pl.ds(v * L, L)]
                rk = kv - kbase
                m = (rk >= 0) & (rk < KP)
                ids16 = lane + (ebase + t * SW + v * L)
                cs = plsc.cumsum(jnp.where(m, 1, 0)) - 1
                idx = jnp.where(m, cn + cs, IDS_DUMP + lane)
                plsc.store_scatter(ids_v, [idx], ids16, mask=m)
                return cn + jnp.max(plsc.all_reduce_population_count(m))
            cnt = lax.fori_loop(0, SVREGS, fv, cnt)
            nfull = cnt // 128
            lax.fori_loop(0, nfull, chunk, 0)
            for j in range(8):
                tmp = ids_v[pl.ds(nfull * 128 + j * L, L)]
                ids_v[pl.ds(j * L, L)] = tmp
            return cnt - nfull * 128

        cnt = lax.fori_loop(0, NSTRIPS, strip_body, jnp.int32(0))
        for t in range(8):
            ids_v[pl.ds(cnt + t * L, L)] = lane + (E + s * 128 + t * L)
        lax.fori_loop(0, (cnt + 127) // 128, chunk, 0)

        plsc.subcore_barrier()
        pltpu.sync_copy(acc.at[pl.ds(s * (KP // NS), KP // NS)],
                        agg_hbm.at[pl.ds(kbase + s * (KP // NS), KP // NS)])
        plsc.subcore_barrier()
        return carry

    lax.fori_loop(0, PASSES, pass_body, 0)


def _sc_aggregate(h, keys_p, src_p, norm_p):
    mesh = plsc.VectorSubcoreMesh(core_axis_name="c", subcore_axis_name="s",
                                  num_cores=NC, num_subcores=NS)
    f = pl.kernel(
        _sc_body,
        out_type=jax.ShapeDtypeStruct((OUT_ROWS, H), jnp.float32),
        mesh=mesh,
        compiler_params=pltpu.CompilerParams(needs_layout_passes=False),
        scratch_types=[
            pltpu.VMEM_SHARED((ACC_ROWS, H), jnp.float32),
            pltpu.VMEM((SW,), jnp.int32),
            pltpu.VMEM((IDS_CAP,), jnp.int32),
            pltpu.VMEM((128,), jnp.int32),
            pltpu.VMEM((128,), jnp.int32),
            pltpu.VMEM((128,), jnp.int32),
            pltpu.VMEM((128,), jnp.float32),
            pltpu.VMEM((128,), jnp.int32),
            pltpu.VMEM((128, H), jnp.float32),
        ],
    )
    return f(h, keys_p, src_p, norm_p)


def _dense_body(agg_ref, h_ref, w2_ref, wself_ref, bias_ref, out_ref):
    acc = jnp.dot(agg_ref[...], w2_ref[...], preferred_element_type=jnp.float32)
    acc += jnp.dot(h_ref[...], wself_ref[...], preferred_element_type=jnp.float32)
    out_ref[...] = jnp.maximum(acc + bias_ref[...], 0.0)


def _dense_stage(aggf, h, w2, w_self, bias2d):
    return pl.pallas_call(
        _dense_body,
        grid=(N // BN,),
        in_specs=[
            pl.BlockSpec((BN, R * H), lambda i: (i, 0)),
            pl.BlockSpec((BN, H), lambda i: (i, 0)),
            pl.BlockSpec((R * H, O), lambda i: (0, 0)),
            pl.BlockSpec((H, O), lambda i: (0, 0)),
            pl.BlockSpec((1, O), lambda i: (0, 0)),
        ],
        out_specs=pl.BlockSpec((BN, O), lambda i: (i, 0)),
        out_shape=jax.ShapeDtypeStruct((N, O), jnp.float32),
    )(aggf, h, w2, w_self, bias2d)


def kernel(h, edge_index, rel_type, norm, W, W_self, bias):
    src = edge_index[0].astype(jnp.int32)
    dst = edge_index[1].astype(jnp.int32)
    key = dst * R + rel_type.astype(jnp.int32)
    normf = norm[:, 0]

    pad = E_PAD - E
    keys_p = jnp.concatenate([key, jnp.full((pad,), BIGKEY, jnp.int32)])
    src_p = jnp.concatenate([src, jnp.zeros((pad,), jnp.int32)])
    norm_p = jnp.concatenate([normf, jnp.zeros((pad,), jnp.float32)])

    agg = _sc_aggregate(h, keys_p, src_p, norm_p)[:N * R]

    aggf = agg.reshape(N, R * H)
    w2 = W.reshape(R * H, O)
    return _dense_stage(aggf, h, w2, W_self, bias.reshape(1, O))
